# baseline (speedup 1.0000x reference)
import functools

import jax
import jax.numpy as jnp
from jax.experimental import pallas as pl
from jax.experimental.pallas import tpu as pltpu


def _conv3_kernel(x_ref, w_ref, b_ref, o_ref, s_ref, g_ref, *, C, W, HW, NB):
    row = jax.lax.broadcasted_iota(jnp.int32, (HW, C), 0)
    col = jax.lax.rem(row, W)
    m_left = (col != 0).astype(jnp.bfloat16)
    m_right = (col != W - 1).astype(jnp.bfloat16)

    for n in range(NB):
        s_ref[n] = x_ref[n].T

    for n in range(NB):
        for ki in range(3):
            for kj in range(3):
                off = ki * W + kj + (W - 1)
                kk = ki * 3 + kj
                sl = s_ref[n, off:off + HW, :]
                if kj == 0:
                    sl = sl * m_left
                elif kj == 2:
                    sl = sl * m_right
                g_ref[n, :, kk * C:(kk + 1) * C] = sl
        acc = jax.lax.dot_general(
            g_ref[n], w_ref[...],
            dimension_numbers=(((1,), (0,)), ((), ())),
            preferred_element_type=jnp.float32)
        o_ref[n] = acc + b_ref[...]


def kernel(x, weight, bias):
    B, C, H, W = x.shape
    O = weight.shape[0]
    HW = H * W
    SW = HW + 4 * W

    xp = jnp.pad(x, ((0, 0), (0, 0), (2, 2), (0, 0)))
    xf = xp.reshape(B, C, SW).astype(jnp.bfloat16)

    w2 = jnp.transpose(weight.reshape(O, C, 9), (2, 1, 0))
    w2 = w2.reshape(9 * C, O).astype(jnp.bfloat16)
    b2 = bias.reshape(1, O).astype(jnp.float32)

    NB = 4
    kfn = functools.partial(_conv3_kernel, C=C, W=W, HW=HW, NB=NB)
    out = pl.pallas_call(
        kfn,
        out_shape=jax.ShapeDtypeStruct((B, HW, O), jnp.float32),
        grid=(B // NB,),
        in_specs=[
            pl.BlockSpec((NB, C, SW), lambda b: (b, 0, 0)),
            pl.BlockSpec((9 * C, O), lambda b: (0, 0)),
            pl.BlockSpec((1, O), lambda b: (0, 0)),
        ],
        out_specs=pl.BlockSpec((NB, HW, O), lambda b: (b, 0, 0)),
        scratch_shapes=[
            pltpu.VMEM((NB, SW, C), jnp.bfloat16),
            pltpu.VMEM((NB, HW, 9 * C), jnp.bfloat16),
        ],
        compiler_params=pltpu.CompilerParams(
            dimension_semantics=("parallel",),
            vmem_limit_bytes=64 * 1024 * 1024,
        ),
    )(xf, w2, b2)

    return out.reshape(-1, O, H, W)


# speedup vs baseline: 1.0910x; 1.0910x over previous
import functools

import jax
import jax.numpy as jnp
from jax.experimental import pallas as pl
from jax.experimental.pallas import tpu as pltpu


def _conv3_kernel(x_ref, w_ref, b_ref, o_ref, s_ref, g_ref, *, C, W, HW, NB):
    pad = 2 * W

    row = jax.lax.broadcasted_iota(jnp.int32, (HW, C), 0)
    col = jax.lax.rem(row, W)
    m_left = (col != 0).astype(jnp.bfloat16)
    m_right = (col != W - 1).astype(jnp.bfloat16)

    for n in range(NB):
        s_ref[n, :pad, :] = jnp.zeros((pad, C), jnp.bfloat16)
        s_ref[n, pad + HW:, :] = jnp.zeros((pad, C), jnp.bfloat16)
        s_ref[n, pad:pad + HW, :] = x_ref[n].astype(jnp.bfloat16).T

    for n in range(NB):
        for ki in range(3):
            for kj in range(3):
                off = ki * W + kj + (W - 1)
                kk = ki * 3 + kj
                sl = s_ref[n, off:off + HW, :]
                if kj == 0:
                    sl = sl * m_left
                elif kj == 2:
                    sl = sl * m_right
                g_ref[n, :, kk * C:(kk + 1) * C] = sl
        acc = jax.lax.dot_general(
            g_ref[n], w_ref[...],
            dimension_numbers=(((1,), (0,)), ((), ())),
            preferred_element_type=jnp.float32)
        o_ref[n] = acc + b_ref[...]


def kernel(x, weight, bias):
    B, C, H, W = x.shape
    O = weight.shape[0]
    HW = H * W

    xf = x.reshape(B, C, HW)
    w2 = jnp.transpose(weight.reshape(O, C, 9), (2, 1, 0))
    w2 = w2.reshape(9 * C, O).astype(jnp.bfloat16)
    b2 = bias.reshape(1, O).astype(jnp.float32)

    NB = 4
    kfn = functools.partial(_conv3_kernel, C=C, W=W, HW=HW, NB=NB)
    out = pl.pallas_call(
        kfn,
        out_shape=jax.ShapeDtypeStruct((B, HW, O), jnp.float32),
        grid=(B // NB,),
        in_specs=[
            pl.BlockSpec((NB, C, HW), lambda b: (b, 0, 0)),
            pl.BlockSpec((9 * C, O), lambda b: (0, 0)),
            pl.BlockSpec((1, O), lambda b: (0, 0)),
        ],
        out_specs=pl.BlockSpec((NB, HW, O), lambda b: (b, 0, 0)),
        scratch_shapes=[
            pltpu.VMEM((NB, HW + 4 * W, C), jnp.bfloat16),
            pltpu.VMEM((NB, HW, 9 * C), jnp.bfloat16),
        ],
        compiler_params=pltpu.CompilerParams(
            dimension_semantics=("parallel",),
            vmem_limit_bytes=64 * 1024 * 1024,
        ),
    )(xf, w2, b2)

    return out.reshape(-1, O, H, W)


# speedup vs baseline: 1.5879x; 1.4555x over previous
import functools

import jax
import jax.numpy as jnp
from jax.experimental import pallas as pl
from jax.experimental.pallas import tpu as pltpu


def _conv3_kernel(x_ref, w_ref, b_ref, o_ref, s_ref, g_ref, *, C, W, HW, NB):
    pad = 2 * W

    row = jax.lax.broadcasted_iota(jnp.int32, (HW, C), 0)
    col = jax.lax.rem(row, W)
    m_left = (col != 0).astype(jnp.bfloat16)
    m_right = (col != W - 1).astype(jnp.bfloat16)

    for n in range(NB):
        s_ref[n, :pad, :] = jnp.zeros((pad, C), jnp.bfloat16)
        s_ref[n, pad + HW:, :] = jnp.zeros((pad, C), jnp.bfloat16)
        s_ref[n, pad:pad + HW, :] = x_ref[n].astype(jnp.bfloat16).T

    for n in range(NB):
        for ki in range(3):
            for kj in range(3):
                off = ki * W + kj + (W - 1)
                kk = ki * 3 + kj
                sl = s_ref[n, off:off + HW, :]
                if kj == 0:
                    sl = sl * m_left
                elif kj == 2:
                    sl = sl * m_right
                g_ref[n, :, kk * C:(kk + 1) * C] = sl
        acc = jax.lax.dot_general(
            g_ref[n], w_ref[...],
            dimension_numbers=(((1,), (0,)), ((), ())),
            preferred_element_type=jnp.float32)
        o_ref[n] = (acc + b_ref[...]).reshape(o_ref.shape[1:])


def kernel(x, weight, bias):
    B, C, H, W = x.shape
    O = weight.shape[0]
    HW = H * W

    xf = x.reshape(B, C, HW)
    w2 = jnp.transpose(weight.reshape(O, C, 9), (2, 1, 0))
    w2 = w2.reshape(9 * C, O).astype(jnp.bfloat16)
    b2 = bias.reshape(1, O).astype(jnp.float32)

    NB = 4
    kfn = functools.partial(_conv3_kernel, C=C, W=W, HW=HW, NB=NB)
    ROWS = HW * O // 1024
    out = pl.pallas_call(
        kfn,
        out_shape=jax.ShapeDtypeStruct((B, ROWS, 8, 128), jnp.float32),
        grid=(B // NB,),
        in_specs=[
            pl.BlockSpec((NB, C, HW), lambda b: (b, 0, 0)),
            pl.BlockSpec((9 * C, O), lambda b: (0, 0)),
            pl.BlockSpec((1, O), lambda b: (0, 0)),
        ],
        out_specs=pl.BlockSpec((NB, ROWS, 8, 128), lambda b: (b, 0, 0, 0)),
        scratch_shapes=[
            pltpu.VMEM((NB, HW + 4 * W, C), jnp.bfloat16),
            pltpu.VMEM((NB, HW, 9 * C), jnp.bfloat16),
        ],
        compiler_params=pltpu.CompilerParams(
            dimension_semantics=("parallel",),
            vmem_limit_bytes=64 * 1024 * 1024,
        ),
    )(xf, w2, b2)

    return out.reshape(-1, O, H, W)
